# baseline (speedup 1.0000x reference)
import functools

import jax
import jax.numpy as jnp
from jax import lax
from jax.experimental import pallas as pl
from jax.experimental.pallas import tpu as pltpu
from jax.experimental.pallas import tpu_sc as plsc

DIM = 64
LANES = 16
NUM_CORES = 2
NUM_SUBCORES = 16
NUM_WORKERS = NUM_CORES * NUM_SUBCORES


def _sc_dot_kernel(n_total: int, chunk: int):
    per_w = n_total // NUM_WORKERS
    n_chunks = per_w // chunk
    mesh = plsc.VectorSubcoreMesh(core_axis_name="c", subcore_axis_name="s")

    @functools.partial(
        pl.kernel,
        out_type=jax.ShapeDtypeStruct((n_total,), jnp.float32),
        mesh=mesh,
        scratch_types=[
            pltpu.VMEM((chunk,), jnp.int32),
            pltpu.VMEM((chunk,), jnp.int32),
            pltpu.VMEM((chunk, DIM), jnp.float32),
            pltpu.VMEM((chunk, DIM), jnp.float32),
            pltpu.VMEM((chunk,), jnp.float32),
            pltpu.SemaphoreType.DMA,
        ],
        compiler_params=pltpu.CompilerParams(
            use_tc_tiling_on_sc=False, needs_layout_passes=False
        ),
    )
    def kern(tgt_hbm, ctx_hbm, tt_hbm, ct_hbm, out_hbm,
             idx_t, idx_c, rows_t, rows_c, out_v, sem):
        wid = lax.axis_index("s") * NUM_CORES + lax.axis_index("c")
        wbase = wid * per_w

        def chunk_body(g, _):
            base = wbase + g * chunk
            pltpu.sync_copy(tgt_hbm.at[pl.ds(base, chunk)], idx_t)
            pltpu.sync_copy(ctx_hbm.at[pl.ds(base, chunk)], idx_c)
            cp_t = pltpu.async_copy(tt_hbm.at[idx_t], rows_t, sem)
            cp_c = pltpu.async_copy(ct_hbm.at[idx_c], rows_c, sem)
            cp_t.wait()
            cp_c.wait()

            def group_body(i, _):
                row0 = i * LANES
                rowv = row0 + lax.iota(jnp.int32, LANES)
                colv = jnp.zeros((LANES,), jnp.int32)
                acc = jnp.zeros((LANES,), jnp.float32)
                for _d in range(DIM):
                    t = plsc.load_gather(rows_t, [rowv, colv])
                    c = plsc.load_gather(rows_c, [rowv, colv])
                    acc = acc + t * c
                    colv = colv + 1
                out_v[pl.ds(row0, LANES)] = acc
                return 0

            lax.fori_loop(0, chunk // LANES, group_body, 0)
            pltpu.sync_copy(out_v, out_hbm.at[pl.ds(base, chunk)])
            return 0

        lax.fori_loop(0, n_chunks, chunk_body, 0)

    return kern


def kernel(target, context, target_table, context_table):
    b, l = target.shape
    n_total = b * l
    tgt = target.reshape(n_total).astype(jnp.int32)
    ctx = context.reshape(n_total).astype(jnp.int32)
    sim = _sc_dot_kernel(n_total, chunk=512)(tgt, ctx, target_table, context_table)
    return sim.reshape(b, l)
